# baseline (speedup 1.0000x reference)
import jax
import jax.numpy as jnp
from jax.experimental import pallas as pl
from jax.experimental.pallas import tpu as pltpu

_E, _CAP, _D = 64, 1024, 512
_H = 4 * _D
_RC = 256
_NC = _CAP // _RC


def _gelu(v):
    return 0.5 * v * (1.0 + jax.lax.erf(v * 0.7071067811865476))


def _ffn_body(sp_ref, x_ref, w1_ref, b1_ref, w2_ref, b2_ref, o_ref):
    e = pl.program_id(0)
    valid = sp_ref[0, e]
    nv = (valid + _RC - 1) // _RC

    o_ref[...] = jnp.zeros_like(o_ref)

    def _chunk(j, carry):
        rows = pl.ds(j * _RC, _RC)
        x = x_ref[0, rows, :]
        h = jnp.dot(x, w1_ref[0], preferred_element_type=jnp.float32)
        h = _gelu(h + b1_ref[0])
        y = jnp.dot(h, w2_ref[0], preferred_element_type=jnp.float32)
        y = y + b2_ref[0]
        ridx = j * _RC + jax.lax.broadcasted_iota(jnp.int32, (_RC, 1), 0)
        o_ref[0, rows, :] = jnp.where(ridx < valid, y, 0.0)
        return carry

    jax.lax.fori_loop(0, nv, _chunk, 0)


def kernel(packed_inputs, valid_load, w1, b1, w2, b2):
    vl = valid_load.astype(jnp.int32)

    eids = jnp.arange(_E, dtype=jnp.int32)
    wmap_row = jax.lax.cummax(jnp.where(vl > 0, eids, 0))
    sp = jnp.stack([vl, wmap_row], axis=0)

    b1r = b1.reshape(_E, 1, _H)
    b2r = b2.reshape(_E, 1, _D)

    def _wmap(e, sp_ref):
        return (sp_ref[1, e], 0, 0)

    out = pl.pallas_call(
        _ffn_body,
        out_shape=jax.ShapeDtypeStruct((_E, _CAP, _D), jnp.float32),
        grid_spec=pltpu.PrefetchScalarGridSpec(
            num_scalar_prefetch=1,
            grid=(_E,),
            in_specs=[
                pl.BlockSpec((1, _CAP, _D), lambda e, sp_ref: (e, 0, 0)),
                pl.BlockSpec((1, _D, _H), _wmap),
                pl.BlockSpec((1, 1, _H), _wmap),
                pl.BlockSpec((1, _H, _D), _wmap),
                pl.BlockSpec((1, 1, _D), _wmap),
            ],
            out_specs=pl.BlockSpec((1, _CAP, _D), lambda e, sp_ref: (e, 0, 0)),
        ),
        compiler_params=pltpu.CompilerParams(
            dimension_semantics=("parallel",),
            vmem_limit_bytes=56 * 1024 * 1024,
        ),
        name="fused_group_ffn",
    )(sp, packed_inputs, w1, b1r, w2, b2r)
    return out
